# baseline (speedup 1.0000x reference)
import functools

import jax
import jax.numpy as jnp
from jax import lax
from jax.experimental import pallas as pl
from jax.experimental.pallas import tpu as pltpu
from jax.experimental.pallas import tpu_sc as plsc

_SIDE = 4096
_N = _SIDE * _SIDE
_K = int(_N * 0.1)
_NC, _NS, _L = 2, 16, 16
_NW = _NC * _NS
_ROWS_W = _SIDE // _NW
_CROWS = 4
_NCHUNK = _ROWS_W // _CROWS
_BINS = 65536


def _total_order_key(u):
    return u ^ lax.shift_right_logical(lax.shift_right_arithmetic(u, 31), 1)


def _sc_stream_shard(k_hbm, bufs, sems, per_vec):
    wid = lax.axis_index("s") * _NC + lax.axis_index("c")
    row0 = wid * _ROWS_W
    kbuf0, kbuf1 = bufs
    sem0, sem1 = sems

    def start(c, kb, sm):
        half = _CROWS // 2
        r = row0 + c * _CROWS
        pltpu.async_copy(k_hbm.at[pl.ds(r, half)], kb.at[pl.ds(0, half)], sm)
        pltpu.async_copy(k_hbm.at[pl.ds(r + half, half)],
                         kb.at[pl.ds(half, half)], sm)

    def wait(kb, sm):
        pltpu.make_async_copy(k_hbm.at[pl.ds(0, _CROWS)], kb, sm).wait()

    def process(kb):
        for j in range(_CROWS):
            @plsc.parallel_loop(0, _SIDE, _L, unroll=16)
            def _(off):
                per_vec(kb[j, pl.ds(off, _L)])

    start(0, kbuf0, sem0)

    def chunk2(c2, _):
        c0 = c2 * 2
        start(c0 + 1, kbuf1, sem1)
        wait(kbuf0, sem0)
        process(kbuf0)

        @pl.when(c0 + 2 < _NCHUNK)
        def _():
            start(c0 + 2, kbuf0, sem0)

        wait(kbuf1, sem1)
        process(kbuf1)
        return 0

    lax.fori_loop(0, _NCHUNK // 2, chunk2, 0)
    return wid


def _zero_hist(hist_v):
    zero16 = jnp.zeros((_L,), jnp.int32)

    @plsc.parallel_loop(0, _BINS, _L, unroll=8)
    def _(i):
        hist_v[pl.ds(i, _L)] = zero16


_SC_SCRATCH = [
    pltpu.VMEM((_CROWS, _SIDE), jnp.int32),
    pltpu.VMEM((_CROWS, _SIDE), jnp.int32),
    pltpu.VMEM((_BINS,), jnp.int32),
    pltpu.SemaphoreType.DMA,
    pltpu.SemaphoreType.DMA,
]
_SC_MESH = plsc.VectorSubcoreMesh(
    core_axis_name="c", subcore_axis_name="s", num_cores=_NC,
    num_subcores=_NS)


def _sc_hist1_body(k_hbm, hist_hbm, kbuf0, kbuf1, hist_v, sem0, sem1):
    _zero_hist(hist_v)
    ones = jnp.ones((_L,), jnp.int32)

    def per_vec(skey):
        plsc.addupdate_scatter(
            hist_v, [lax.shift_right_logical(skey, 16)], ones)

    wid = _sc_stream_shard(k_hbm, (kbuf0, kbuf1), (sem0, sem1), per_vec)
    pltpu.sync_copy(hist_v, hist_hbm.at[wid])


def _sc_hist2_body(cfg_hbm, k_hbm, hist_hbm, cfg_v, kbuf0, kbuf1, hist_v,
                   sem0, sem1):
    _zero_hist(hist_v)
    pltpu.sync_copy(cfg_hbm, cfg_v)
    bv = cfg_v[pl.ds(0, _L)]
    ones = jnp.ones((_L,), jnp.int32)

    def per_vec(skey):
        high = lax.shift_right_arithmetic(skey, 16)
        low = skey & 0xFFFF
        plsc.addupdate_scatter(hist_v, [low], ones, mask=high == bv)

    wid = _sc_stream_shard(k_hbm, (kbuf0, kbuf1), (sem0, sem1), per_vec)
    pltpu.sync_copy(hist_v, hist_hbm.at[wid])


_sc_hist1 = functools.partial(
    pl.kernel,
    out_type=jax.ShapeDtypeStruct((_NW, _BINS), jnp.int32),
    mesh=_SC_MESH,
    scratch_types=_SC_SCRATCH,
    compiler_params=pltpu.CompilerParams(needs_layout_passes=False),
)(_sc_hist1_body)

_sc_hist2 = functools.partial(
    pl.kernel,
    out_type=jax.ShapeDtypeStruct((_NW, _BINS), jnp.int32),
    mesh=_SC_MESH,
    scratch_types=[pltpu.VMEM((_L,), jnp.int32)] + _SC_SCRATCH,
    compiler_params=pltpu.CompilerParams(needs_layout_passes=False),
)(_sc_hist2_body)


def _tc_key_body(f_ref, r_ref, k_ref):
    d = f_ref[...] - r_ref[...]
    k_ref[...] = _total_order_key(lax.bitcast_convert_type(d, jnp.int32))


_ROWS_PER_BLK = 256
_tc_key = pl.pallas_call(
    _tc_key_body,
    grid=(_SIDE // _ROWS_PER_BLK,),
    in_specs=[pl.BlockSpec((_ROWS_PER_BLK, _SIDE), lambda i: (i, 0))] * 2,
    out_specs=pl.BlockSpec((_ROWS_PER_BLK, _SIDE), lambda i: (i, 0)),
    out_shape=jax.ShapeDtypeStruct((_SIDE, _SIDE), jnp.int32),
)


def _tc_mask_body(thr_ref, k_ref, w_ref, o_ref):
    o_ref[...] = jnp.where(k_ref[...] >= thr_ref[0], jnp.float32(0.0),
                           w_ref[...])


_tc_mask = pl.pallas_call(
    _tc_mask_body,
    grid_spec=pltpu.PrefetchScalarGridSpec(
        num_scalar_prefetch=1,
        grid=(_SIDE // _ROWS_PER_BLK,),
        in_specs=[pl.BlockSpec((_ROWS_PER_BLK, _SIDE), lambda i, s: (i, 0))] * 2,
        out_specs=pl.BlockSpec((_ROWS_PER_BLK, _SIDE), lambda i, s: (i, 0)),
    ),
    out_shape=jax.ShapeDtypeStruct((_SIDE, _SIDE), jnp.float32),
)


def _suffix_rank(hist, rank):
    csum = jnp.cumsum(hist[::-1])
    below = csum < rank
    i = jnp.sum(below, dtype=jnp.int32)
    b = jnp.int32(_BINS - 1) - i
    above = jnp.max(jnp.where(below, csum, 0))
    return b, above


def kernel(forget_hess, retain_hess, weights):
    skey = _tc_key(forget_hess, retain_hess)

    h1 = jnp.roll(_sc_hist1(skey).sum(axis=0), _BINS // 2)
    b, above1 = _suffix_rank(h1, _K)
    k2 = _K - above1

    cfg2 = jnp.full((_L,), b - jnp.int32(32768), jnp.int32)
    h2 = _sc_hist2(cfg2, skey).sum(axis=0)
    lt, _ = _suffix_rank(h2, k2)

    thr = (b - jnp.int32(32768)) * jnp.int32(65536) + lt
    thr_arr = jnp.full((1,), thr, jnp.int32)
    return _tc_mask(thr_arr, skey, weights)


# speedup vs baseline: 1.0600x; 1.0600x over previous
import functools

import jax
import jax.numpy as jnp
from jax import lax
from jax.experimental import pallas as pl
from jax.experimental.pallas import tpu as pltpu
from jax.experimental.pallas import tpu_sc as plsc

_SIDE = 4096
_N = _SIDE * _SIDE
_K = int(_N * 0.1)
_NC, _NS, _L = 2, 16, 16
_NW = _NC * _NS
_ROWS_W = _SIDE // _NW
_CROWS = 4
_NCHUNK = _ROWS_W // _CROWS
_BINS = 65536


def _total_order_key(u):
    return u ^ lax.shift_right_logical(lax.shift_right_arithmetic(u, 31), 1)


def _sc_stream_shard(k_hbm, bufs, sems, per_vec):
    wid = lax.axis_index("s") * _NC + lax.axis_index("c")
    row0 = wid * _ROWS_W
    kbuf0, kbuf1 = bufs
    sem0, sem1 = sems

    def start(c, kb, sm):
        pltpu.async_copy(k_hbm.at[pl.ds(row0 + c * _CROWS, _CROWS)], kb, sm)

    def wait(kb, sm):
        pltpu.make_async_copy(k_hbm.at[pl.ds(0, _CROWS)], kb, sm).wait()

    def process(kb):
        for j in range(_CROWS):
            @plsc.parallel_loop(0, _SIDE, _L, unroll=16)
            def _(off):
                per_vec(kb[j, pl.ds(off, _L)])

    start(0, kbuf0, sem0)

    def chunk2(c2, _):
        c0 = c2 * 2
        start(c0 + 1, kbuf1, sem1)
        wait(kbuf0, sem0)
        process(kbuf0)

        @pl.when(c0 + 2 < _NCHUNK)
        def _():
            start(c0 + 2, kbuf0, sem0)

        wait(kbuf1, sem1)
        process(kbuf1)
        return 0

    lax.fori_loop(0, _NCHUNK // 2, chunk2, 0)
    return wid


def _zero_hist(hist_v):
    zero16 = jnp.zeros((_L,), jnp.int32)

    @plsc.parallel_loop(0, _BINS, _L, unroll=8)
    def _(i):
        hist_v[pl.ds(i, _L)] = zero16


_SC_SCRATCH = [
    pltpu.VMEM((_CROWS, _SIDE), jnp.int32),
    pltpu.VMEM((_CROWS, _SIDE), jnp.int32),
    pltpu.VMEM((_BINS,), jnp.int32),
    pltpu.SemaphoreType.DMA,
    pltpu.SemaphoreType.DMA,
]
_SC_MESH = plsc.VectorSubcoreMesh(
    core_axis_name="c", subcore_axis_name="s", num_cores=_NC,
    num_subcores=_NS)


def _sc_hist1_body(k_hbm, hist_hbm, kbuf0, kbuf1, hist_v, sem0, sem1):
    _zero_hist(hist_v)
    ones = jnp.ones((_L,), jnp.int32)

    def per_vec(skey):
        plsc.addupdate_scatter(
            hist_v, [lax.shift_right_logical(skey, 16)], ones)

    wid = _sc_stream_shard(k_hbm, (kbuf0, kbuf1), (sem0, sem1), per_vec)
    pltpu.sync_copy(hist_v, hist_hbm.at[wid])


def _sc_hist2_body(cfg_hbm, k_hbm, hist_hbm, cfg_v, kbuf0, kbuf1, hist_v,
                   sem0, sem1):
    _zero_hist(hist_v)
    pltpu.sync_copy(cfg_hbm, cfg_v)
    bv = cfg_v[pl.ds(0, _L)]
    ones = jnp.ones((_L,), jnp.int32)

    def per_vec(skey):
        high = lax.shift_right_arithmetic(skey, 16)
        low = skey & 0xFFFF
        plsc.addupdate_scatter(hist_v, [low], ones, mask=high == bv)

    wid = _sc_stream_shard(k_hbm, (kbuf0, kbuf1), (sem0, sem1), per_vec)
    pltpu.sync_copy(hist_v, hist_hbm.at[wid])


_sc_hist1 = functools.partial(
    pl.kernel,
    out_type=jax.ShapeDtypeStruct((_NW, _BINS), jnp.int32),
    mesh=_SC_MESH,
    scratch_types=_SC_SCRATCH,
    compiler_params=pltpu.CompilerParams(needs_layout_passes=False),
)(_sc_hist1_body)

_sc_hist2 = functools.partial(
    pl.kernel,
    out_type=jax.ShapeDtypeStruct((_NW, _BINS), jnp.int32),
    mesh=_SC_MESH,
    scratch_types=[pltpu.VMEM((_L,), jnp.int32)] + _SC_SCRATCH,
    compiler_params=pltpu.CompilerParams(needs_layout_passes=False),
)(_sc_hist2_body)


def _tc_key_body(f_ref, r_ref, k_ref):
    d = f_ref[...] - r_ref[...]
    k_ref[...] = _total_order_key(lax.bitcast_convert_type(d, jnp.int32))


_ROWS_PER_BLK = 256
_tc_key = pl.pallas_call(
    _tc_key_body,
    grid=(_SIDE // _ROWS_PER_BLK,),
    in_specs=[pl.BlockSpec((_ROWS_PER_BLK, _SIDE), lambda i: (i, 0))] * 2,
    out_specs=pl.BlockSpec((_ROWS_PER_BLK, _SIDE), lambda i: (i, 0)),
    out_shape=jax.ShapeDtypeStruct((_SIDE, _SIDE), jnp.int32),
)


def _tc_mask_body(thr_ref, k_ref, w_ref, o_ref):
    o_ref[...] = jnp.where(k_ref[...] >= thr_ref[0], jnp.float32(0.0),
                           w_ref[...])


_tc_mask = pl.pallas_call(
    _tc_mask_body,
    grid_spec=pltpu.PrefetchScalarGridSpec(
        num_scalar_prefetch=1,
        grid=(_SIDE // _ROWS_PER_BLK,),
        in_specs=[pl.BlockSpec((_ROWS_PER_BLK, _SIDE), lambda i, s: (i, 0))] * 2,
        out_specs=pl.BlockSpec((_ROWS_PER_BLK, _SIDE), lambda i, s: (i, 0)),
    ),
    out_shape=jax.ShapeDtypeStruct((_SIDE, _SIDE), jnp.float32),
)


def _suffix_rank(hist, rank):
    csum = jnp.cumsum(hist[::-1])
    below = csum < rank
    i = jnp.sum(below, dtype=jnp.int32)
    b = jnp.int32(_BINS - 1) - i
    above = jnp.max(jnp.where(below, csum, 0))
    return b, above


def kernel(forget_hess, retain_hess, weights):
    skey = _tc_key(forget_hess, retain_hess)

    h1 = jnp.roll(_sc_hist1(skey).sum(axis=0), _BINS // 2)
    b, above1 = _suffix_rank(h1, _K)
    k2 = _K - above1

    cfg2 = jnp.full((_L,), b - jnp.int32(32768), jnp.int32)
    h2 = _sc_hist2(cfg2, skey).sum(axis=0)
    lt, _ = _suffix_rank(h2, k2)

    thr = (b - jnp.int32(32768)) * jnp.int32(65536) + lt
    thr_arr = jnp.full((1,), thr, jnp.int32)
    return _tc_mask(thr_arr, skey, weights)
